# baseline (speedup 1.0000x reference)
import functools

import jax
import jax.numpy as jnp
from jax import lax
from jax.experimental import pallas as pl
from jax.experimental.pallas import tpu as pltpu
from jax.experimental.pallas import tpu_sc as plsc

N = 10000
E = 320000
D = 128

NC = 2
NS = 16
NW = NC * NS
EPW = E // NW
B = 128
NB = 80
EPAD = NW * NB * B
PADW = (EPAD - E) // NW
NTRASH = N + 128
CH = 8
SUP = 16
NSUP = NB // SUP

ROWS_A = 624
TAIL0 = NS * ROWS_A
TAIL = N - TAIL0


def _sc_mesh():
    return plsc.VectorSubcoreMesh(core_axis_name="c", subcore_axis_name="s")


def _tilewise_copy(src, dst, sid):
    r0 = sid * ROWS_A
    pltpu.sync_copy(src.at[pl.ds(r0, ROWS_A)], dst.at[pl.ds(r0, ROWS_A)])

    @pl.when(sid == NS - 1)
    def _():
        pltpu.sync_copy(src.at[pl.ds(TAIL0, TAIL)], dst.at[pl.ds(TAIL0, TAIL)])


@functools.partial(
    pl.kernel,
    out_type=jax.ShapeDtypeStruct((NC * N,), jnp.float32),
    mesh=_sc_mesh(),
    scratch_types=[
        pltpu.VMEM((NB, B), jnp.int32),
        pltpu.VMEM((B,), jnp.float32),
        pltpu.VMEM((ROWS_A,), jnp.float32),
        pltpu.VMEM_SHARED((NTRASH,), jnp.float32),
    ],
)
def _deg_kernel(ei3_hbm, ones_hbm, zeros_hbm, out_hbm, didx_v, ones_v, vbuf,
                acc_sh):
    cid = lax.axis_index("c")
    sid = lax.axis_index("s")
    wid = cid * NS + sid
    r0 = sid * ROWS_A
    pltpu.sync_copy(zeros_hbm, vbuf)
    pltpu.sync_copy(vbuf, acc_sh.at[pl.ds(r0, ROWS_A)])

    @pl.when(sid == NS - 1)
    def _():
        pltpu.sync_copy(vbuf.at[pl.ds(0, TAIL + NTRASH - N)],
                        acc_sh.at[pl.ds(TAIL0, TAIL + NTRASH - N)])

    pltpu.sync_copy(ones_hbm, ones_v)
    pltpu.sync_copy(ei3_hbm.at[1, wid], didx_v)
    plsc.subcore_barrier()

    def body(j, carry):
        pltpu.sync_copy(ones_v, acc_sh.at[didx_v.at[j]], add=True)
        return carry

    lax.fori_loop(0, NB, body, 0)
    plsc.subcore_barrier()
    o0 = cid * N + r0
    pltpu.sync_copy(acc_sh.at[pl.ds(r0, ROWS_A)], vbuf)
    pltpu.sync_copy(vbuf, out_hbm.at[pl.ds(o0, ROWS_A)])

    @pl.when(sid == NS - 1)
    def _():
        pltpu.sync_copy(acc_sh.at[pl.ds(TAIL0, TAIL)], vbuf.at[pl.ds(0, TAIL)])
        pltpu.sync_copy(vbuf.at[pl.ds(0, TAIL)],
                        out_hbm.at[pl.ds(cid * N + TAIL0, TAIL)])


@functools.partial(
    pl.kernel,
    out_type=jax.ShapeDtypeStruct((NC, N, D), jnp.float32),
    mesh=_sc_mesh(),
    scratch_types=[
        pltpu.VMEM((NB, B), jnp.int32),
        pltpu.VMEM((CH, B), jnp.int32),
        pltpu.VMEM((CH, B), jnp.int32),
        pltpu.VMEM((B, D), jnp.float32),
        pltpu.VMEM((B, D), jnp.float32),
        pltpu.VMEM_SHARED((NTRASH, D), jnp.float32),
        pltpu.SemaphoreType.DMA,
        pltpu.SemaphoreType.DMA,
        pltpu.SemaphoreType.DMA,
        pltpu.SemaphoreType.DMA,
        pltpu.SemaphoreType.DMA,
    ],
)
def _agg_kernel(g_hbm, ei3_hbm, zeros_hbm, out_hbm,
                sidx_v, dc0, dc1, rows0, rows1, acc_sh,
                g_sem0, g_sem1, s_sem0, s_sem1, i_sem):
    cid = lax.axis_index("c")
    sid = lax.axis_index("s")
    wid = cid * NS + sid
    r0_ = sid * ROWS_A
    pltpu.async_copy(zeros_hbm, acc_sh.at[pl.ds(r0_, ROWS_A)], s_sem0)
    pltpu.async_copy(ei3_hbm.at[0, wid], sidx_v, g_sem0)
    pltpu.async_copy(ei3_hbm.at[1, wid, pl.ds(0, CH)], dc0, g_sem1)

    @pl.when(sid == NS - 1)
    def _():
        pltpu.async_copy(zeros_hbm.at[pl.ds(0, TAIL + NTRASH - N)],
                         acc_sh.at[pl.ds(TAIL0, TAIL + NTRASH - N)], s_sem1)
        pltpu.make_async_copy(
            zeros_hbm.at[pl.ds(0, TAIL + NTRASH - N)],
            acc_sh.at[pl.ds(TAIL0, TAIL + NTRASH - N)], s_sem1).wait()

    pltpu.make_async_copy(zeros_hbm, acc_sh.at[pl.ds(r0_, ROWS_A)],
                          s_sem0).wait()
    pltpu.make_async_copy(ei3_hbm.at[0, wid], sidx_v, g_sem0).wait()
    pltpu.make_async_copy(ei3_hbm.at[1, wid, pl.ds(0, CH)], dc0,
                          g_sem1).wait()
    plsc.subcore_barrier()

    rows = (rows0, rows1)
    dcs = (dc0, dc1)
    g_sems = (g_sem0, g_sem1)
    s_sems = (s_sem0, s_sem1)

    def wait_g(p, j):
        pltpu.make_async_copy(g_hbm.at[sidx_v.at[j]], rows[p], g_sems[p]).wait()

    def wait_s(p):
        pltpu.make_async_copy(rows[p], acc_sh.at[dc0.at[0]], s_sems[p]).wait()

    def wait_i(dc):
        pltpu.make_async_copy(ei3_hbm.at[1, wid, pl.ds(0, CH)], dc,
                              i_sem).wait()

    pltpu.async_copy(g_hbm.at[sidx_v.at[0]], rows0, g_sem0)

    def body(s, carry):
        for q in range(SUP // 2):
            k = 2 * q
            j = SUP * s + k
            pc = (k // CH) % 2
            r = k % CH
            if q == 0:
                @pl.when(s > 0)
                def _():
                    wait_i(dc0)
            if k == CH:
                wait_i(dc1)
            wait_g(0, j)
            pltpu.async_copy(rows0, acc_sh.at[dcs[pc].at[r]], s_sems[0],
                             add=True)
            if q == 0:
                @pl.when(s > 0)
                def _():
                    wait_s(1)
            else:
                wait_s(1)
            if q == 0:
                pltpu.async_copy(
                    ei3_hbm.at[1, wid, pl.ds((2 * s + 1) * CH, CH)], dc1,
                    i_sem)
            if q == CH // 2:
                @pl.when(s < NSUP - 1)
                def _():
                    pltpu.async_copy(
                        ei3_hbm.at[1, wid, pl.ds((2 * s + 2) * CH, CH)], dc0,
                        i_sem)
            pltpu.async_copy(g_hbm.at[sidx_v.at[j + 1]], rows1, g_sems[1])
            wait_g(1, j + 1)
            pltpu.async_copy(rows1, acc_sh.at[dcs[pc].at[r + 1]], s_sems[1],
                             add=True)
            wait_s(0)
            @pl.when(j + 2 < NB)
            def _():
                pltpu.async_copy(g_hbm.at[sidx_v.at[j + 2]], rows0, g_sems[0])

        return carry

    lax.fori_loop(0, NSUP, body, 0)
    wait_s(1)
    plsc.subcore_barrier()
    _tilewise_copy(acc_sh, out_hbm.at[cid], sid)


_RB = 1000


def _dis_block(d_ref):
    deg = d_ref[...] + 1.0
    return lax.rsqrt(deg)


def _tc_mm_body(x_ref, w_ref, p_ref):
    p_ref[...] = jnp.dot(x_ref[...], w_ref[...],
                         preferred_element_type=jnp.float32)


def _tc_scale_body(p_ref, d_ref, g0_ref):
    g0_ref[...] = p_ref[...] * _dis_block(d_ref)


def _tc_mid_body(s0_ref, s1_ref, g_ref, d_ref, b_ref, w_ref, out_ref):
    dis = _dis_block(d_ref)
    t = dis * (s0_ref[0] + s1_ref[0] + g_ref[...]) + b_ref[...]
    h = jnp.where(t >= 0.0, t, 0.01 * t)
    out_ref[...] = jnp.dot(h, w_ref[...], preferred_element_type=jnp.float32) * dis


def _tc_last_body(s0_ref, s1_ref, g_ref, d_ref, b_ref, out_ref):
    dis = _dis_block(d_ref)
    t = dis * (s0_ref[0] + s1_ref[0] + g_ref[...]) + b_ref[...]
    out_ref[...] = jnp.where(t >= 0.0, t, 0.01 * t)


_row_spec = pl.BlockSpec((_RB, D), lambda i: (i, 0))
_deg_spec = pl.BlockSpec((_RB, 1), lambda i: (i, 0))
_w_spec = pl.BlockSpec((D, D), lambda i: (0, 0))
_b_spec = pl.BlockSpec((1, D), lambda i: (0, 0))
_s0_spec = pl.BlockSpec((1, _RB, D), lambda i: (0, i, 0))
_s1_spec = pl.BlockSpec((1, _RB, D), lambda i: (1, i, 0))
_out_struct = jax.ShapeDtypeStruct((N, D), jnp.float32)

_tc_mm = pl.pallas_call(
    _tc_mm_body, grid=(N // _RB,),
    in_specs=[_row_spec, _w_spec],
    out_specs=_row_spec, out_shape=_out_struct)

_tc_scale = pl.pallas_call(
    _tc_scale_body, grid=(N // _RB,),
    in_specs=[_row_spec, _deg_spec],
    out_specs=_row_spec, out_shape=_out_struct)

_tc_mid = pl.pallas_call(
    _tc_mid_body, grid=(N // _RB,),
    in_specs=[_s0_spec, _s1_spec, _row_spec, _deg_spec, _b_spec, _w_spec],
    out_specs=_row_spec, out_shape=_out_struct)

_tc_last = pl.pallas_call(
    _tc_last_body, grid=(N // _RB,),
    in_specs=[_s0_spec, _s1_spec, _row_spec, _deg_spec, _b_spec],
    out_specs=_row_spec, out_shape=_out_struct)


def kernel(x, edge_index, W0, b0, W1, b1):
    ei = edge_index.astype(jnp.int32).reshape(2, NW, EPW)
    padw = jnp.arange(PADW, dtype=jnp.int32)
    pads = jnp.stack([
        jnp.broadcast_to((padw * 37) % N, (NW, PADW)),
        jnp.broadcast_to(N + padw % (NTRASH - N), (NW, PADW)),
    ])
    ei3 = jnp.concatenate([ei, pads], axis=2).reshape(2, NW, NB, B)
    ones1 = jnp.ones((B,), jnp.float32)
    zeros1 = jnp.zeros((ROWS_A,), jnp.float32)
    zeros_sm = jnp.zeros((ROWS_A, D), jnp.float32)

    p0 = _tc_mm(x, W0)
    degp = _deg_kernel(ei3, ones1, zeros1)
    dd = (degp[:N] + degp[N:]).reshape(N, 1)

    g0 = _tc_scale(p0, dd)
    s0 = _agg_kernel(g0, ei3, zeros_sm)
    g1 = _tc_mid(s0, s0, g0, dd, b0.reshape(1, D), W1)
    s1 = _agg_kernel(g1, ei3, zeros_sm)
    return _tc_last(s1, s1, g1, dd, b1.reshape(1, D))


# speedup vs baseline: 1.0238x; 1.0238x over previous
import functools

import jax
import jax.numpy as jnp
from jax import lax
from jax.experimental import pallas as pl
from jax.experimental.pallas import tpu as pltpu
from jax.experimental.pallas import tpu_sc as plsc

N = 10000
E = 320000
D = 128

NC = 2
NS = 16
NW = NC * NS
EPW = E // NW
B = 128
NB = 80
EPAD = NW * NB * B
PADW = (EPAD - E) // NW
NTRASH = N + 128
CH = 8
SUP = 16
NSUP = NB // SUP

ROWS_A = 624
TAIL0 = NS * ROWS_A
TAIL = N - TAIL0


def _sc_mesh():
    return plsc.VectorSubcoreMesh(core_axis_name="c", subcore_axis_name="s")


def _tilewise_copy(src, dst, sid):
    r0 = sid * ROWS_A
    pltpu.sync_copy(src.at[pl.ds(r0, ROWS_A)], dst.at[pl.ds(r0, ROWS_A)])

    @pl.when(sid == NS - 1)
    def _():
        pltpu.sync_copy(src.at[pl.ds(TAIL0, TAIL)], dst.at[pl.ds(TAIL0, TAIL)])


@functools.partial(
    pl.kernel,
    out_type=jax.ShapeDtypeStruct((NC * N,), jnp.float32),
    mesh=_sc_mesh(),
    scratch_types=[
        pltpu.VMEM((NB, B), jnp.int32),
        pltpu.VMEM((B,), jnp.float32),
        pltpu.VMEM((ROWS_A,), jnp.float32),
        pltpu.VMEM_SHARED((NTRASH,), jnp.float32),
    ],
)
def _deg_kernel(ei3_hbm, ones_hbm, zeros_hbm, out_hbm, didx_v, ones_v, vbuf,
                acc_sh):
    cid = lax.axis_index("c")
    sid = lax.axis_index("s")
    wid = cid * NS + sid
    r0 = sid * ROWS_A
    pltpu.sync_copy(zeros_hbm, vbuf)
    pltpu.sync_copy(vbuf, acc_sh.at[pl.ds(r0, ROWS_A)])

    @pl.when(sid == NS - 1)
    def _():
        pltpu.sync_copy(vbuf.at[pl.ds(0, TAIL + NTRASH - N)],
                        acc_sh.at[pl.ds(TAIL0, TAIL + NTRASH - N)])

    pltpu.sync_copy(ones_hbm, ones_v)
    pltpu.sync_copy(ei3_hbm.at[1, wid], didx_v)
    plsc.subcore_barrier()

    def body(j, carry):
        pltpu.sync_copy(ones_v, acc_sh.at[didx_v.at[j]], add=True)
        return carry

    lax.fori_loop(0, NB, body, 0)
    plsc.subcore_barrier()
    o0 = cid * N + r0
    pltpu.sync_copy(acc_sh.at[pl.ds(r0, ROWS_A)], vbuf)
    pltpu.sync_copy(vbuf, out_hbm.at[pl.ds(o0, ROWS_A)])

    @pl.when(sid == NS - 1)
    def _():
        pltpu.sync_copy(acc_sh.at[pl.ds(TAIL0, TAIL)], vbuf.at[pl.ds(0, TAIL)])
        pltpu.sync_copy(vbuf.at[pl.ds(0, TAIL)],
                        out_hbm.at[pl.ds(cid * N + TAIL0, TAIL)])


@functools.partial(
    pl.kernel,
    out_type=jax.ShapeDtypeStruct((NC, N, D), jnp.float32),
    mesh=_sc_mesh(),
    scratch_types=[
        pltpu.VMEM((NB, B), jnp.int32),
        pltpu.VMEM((CH, B), jnp.int32),
        pltpu.VMEM((CH, B), jnp.int32),
        pltpu.VMEM((B, D), jnp.float32),
        pltpu.VMEM((B, D), jnp.float32),
        pltpu.VMEM_SHARED((NTRASH, D), jnp.float32),
        pltpu.SemaphoreType.DMA,
        pltpu.SemaphoreType.DMA,
        pltpu.SemaphoreType.DMA,
        pltpu.SemaphoreType.DMA,
        pltpu.SemaphoreType.DMA,
    ],
)
def _agg_kernel(g_hbm, ei3_hbm, zeros_hbm, out_hbm,
                sidx_v, dc0, dc1, rows0, rows1, acc_sh,
                g_sem0, g_sem1, s_sem0, s_sem1, i_sem):
    cid = lax.axis_index("c")
    sid = lax.axis_index("s")
    wid = cid * NS + sid
    r0_ = sid * ROWS_A
    pltpu.async_copy(zeros_hbm, acc_sh.at[pl.ds(r0_, ROWS_A)], s_sem0)
    pltpu.async_copy(ei3_hbm.at[0, wid], sidx_v, g_sem0)
    pltpu.async_copy(ei3_hbm.at[1, wid, pl.ds(0, CH)], dc0, g_sem1)

    @pl.when(sid == NS - 1)
    def _():
        pltpu.async_copy(zeros_hbm.at[pl.ds(0, TAIL + NTRASH - N)],
                         acc_sh.at[pl.ds(TAIL0, TAIL + NTRASH - N)], s_sem1)
        pltpu.make_async_copy(
            zeros_hbm.at[pl.ds(0, TAIL + NTRASH - N)],
            acc_sh.at[pl.ds(TAIL0, TAIL + NTRASH - N)], s_sem1).wait()

    pltpu.make_async_copy(zeros_hbm, acc_sh.at[pl.ds(r0_, ROWS_A)],
                          s_sem0).wait()
    pltpu.make_async_copy(ei3_hbm.at[0, wid], sidx_v, g_sem0).wait()
    pltpu.make_async_copy(ei3_hbm.at[1, wid, pl.ds(0, CH)], dc0,
                          g_sem1).wait()
    plsc.subcore_barrier()

    rows = (rows0, rows1)
    dcs = (dc0, dc1)
    g_sems = (g_sem0, g_sem1)
    s_sems = (s_sem0, s_sem1)

    def wait_g(p, j):
        pltpu.make_async_copy(g_hbm.at[sidx_v.at[j]], rows[p], g_sems[p]).wait()

    def wait_s(p):
        pltpu.make_async_copy(rows[p], acc_sh.at[dc0.at[0]], s_sems[p]).wait()

    def wait_i(dc):
        pltpu.make_async_copy(ei3_hbm.at[1, wid, pl.ds(0, CH)], dc,
                              i_sem).wait()

    pltpu.async_copy(g_hbm.at[sidx_v.at[0]], rows0, g_sem0)

    def body(s, carry):
        for q in range(SUP // 2):
            k = 2 * q
            j = SUP * s + k
            pc = (k // CH) % 2
            r = k % CH
            if q == 0:
                @pl.when(s > 0)
                def _():
                    wait_i(dc0)
            if k == CH:
                wait_i(dc1)
            wait_g(0, j)
            pltpu.async_copy(rows0, acc_sh.at[dcs[pc].at[r]], s_sems[0],
                             add=True)
            if q == 0:
                @pl.when(s > 0)
                def _():
                    wait_s(1)
            else:
                wait_s(1)
            if q == 0:
                pltpu.async_copy(
                    ei3_hbm.at[1, wid, pl.ds((2 * s + 1) * CH, CH)], dc1,
                    i_sem)
            if q == CH // 2:
                @pl.when(s < NSUP - 1)
                def _():
                    pltpu.async_copy(
                        ei3_hbm.at[1, wid, pl.ds((2 * s + 2) * CH, CH)], dc0,
                        i_sem)
            pltpu.async_copy(g_hbm.at[sidx_v.at[j + 1]], rows1, g_sems[1])
            wait_g(1, j + 1)
            pltpu.async_copy(rows1, acc_sh.at[dcs[pc].at[r + 1]], s_sems[1],
                             add=True)
            wait_s(0)
            @pl.when(j + 2 < NB)
            def _():
                pltpu.async_copy(g_hbm.at[sidx_v.at[j + 2]], rows0, g_sems[0])

        return carry

    lax.fori_loop(0, NSUP, body, 0)
    wait_s(1)
    plsc.subcore_barrier()
    _tilewise_copy(acc_sh, out_hbm.at[cid], sid)


_RB = 5000


def _dis_block(d_ref):
    deg = d_ref[...] + 1.0
    return lax.rsqrt(deg)


def _tc_mm_body(x_ref, w_ref, p_ref):
    p_ref[...] = jnp.dot(x_ref[...], w_ref[...],
                         preferred_element_type=jnp.float32)


def _tc_scale_body(p_ref, d_ref, g0_ref):
    g0_ref[...] = p_ref[...] * _dis_block(d_ref)


def _tc_mid_body(s0_ref, s1_ref, g_ref, d_ref, b_ref, w_ref, out_ref):
    dis = _dis_block(d_ref)
    t = dis * (s0_ref[0] + s1_ref[0] + g_ref[...]) + b_ref[...]
    h = jnp.where(t >= 0.0, t, 0.01 * t)
    out_ref[...] = jnp.dot(h, w_ref[...], preferred_element_type=jnp.float32) * dis


def _tc_last_body(s0_ref, s1_ref, g_ref, d_ref, b_ref, out_ref):
    dis = _dis_block(d_ref)
    t = dis * (s0_ref[0] + s1_ref[0] + g_ref[...]) + b_ref[...]
    out_ref[...] = jnp.where(t >= 0.0, t, 0.01 * t)


_row_spec = pl.BlockSpec((_RB, D), lambda i: (i, 0))
_deg_spec = pl.BlockSpec((_RB, 1), lambda i: (i, 0))
_w_spec = pl.BlockSpec((D, D), lambda i: (0, 0))
_b_spec = pl.BlockSpec((1, D), lambda i: (0, 0))
_s0_spec = pl.BlockSpec((1, _RB, D), lambda i: (0, i, 0))
_s1_spec = pl.BlockSpec((1, _RB, D), lambda i: (1, i, 0))
_out_struct = jax.ShapeDtypeStruct((N, D), jnp.float32)

_tc_mm = pl.pallas_call(
    _tc_mm_body, grid=(N // _RB,),
    in_specs=[_row_spec, _w_spec],
    out_specs=_row_spec, out_shape=_out_struct)

_tc_scale = pl.pallas_call(
    _tc_scale_body, grid=(N // _RB,),
    in_specs=[_row_spec, _deg_spec],
    out_specs=_row_spec, out_shape=_out_struct)

_tc_mid = pl.pallas_call(
    _tc_mid_body, grid=(N // _RB,),
    in_specs=[_s0_spec, _s1_spec, _row_spec, _deg_spec, _b_spec, _w_spec],
    out_specs=_row_spec, out_shape=_out_struct)

_tc_last = pl.pallas_call(
    _tc_last_body, grid=(N // _RB,),
    in_specs=[_s0_spec, _s1_spec, _row_spec, _deg_spec, _b_spec],
    out_specs=_row_spec, out_shape=_out_struct)


def kernel(x, edge_index, W0, b0, W1, b1):
    ei = edge_index.astype(jnp.int32).reshape(2, NW, EPW)
    padw = jnp.arange(PADW, dtype=jnp.int32)
    pads = jnp.stack([
        jnp.broadcast_to((padw * 37) % N, (NW, PADW)),
        jnp.broadcast_to(N + padw % (NTRASH - N), (NW, PADW)),
    ])
    ei3 = jnp.concatenate([ei, pads], axis=2).reshape(2, NW, NB, B)
    ones1 = jnp.ones((B,), jnp.float32)
    zeros1 = jnp.zeros((ROWS_A,), jnp.float32)
    zeros_sm = jnp.zeros((ROWS_A, D), jnp.float32)

    p0 = _tc_mm(x, W0)
    degp = _deg_kernel(ei3, ones1, zeros1)
    dd = (degp[:N] + degp[N:]).reshape(N, 1)

    g0 = _tc_scale(p0, dd)
    s0 = _agg_kernel(g0, ei3, zeros_sm)
    g1 = _tc_mid(s0, s0, g0, dd, b0.reshape(1, D), W1)
    s1 = _agg_kernel(g1, ei3, zeros_sm)
    return _tc_last(s1, s1, g1, dd, b1.reshape(1, D))
